# Initial kernel scaffold; baseline (speedup 1.0000x reference)
import jax
import jax.numpy as jnp
from jax.experimental import pallas as pl


def kernel(z, params):
    raise NotImplementedError("write your pallas kernel here")


# speedup vs baseline: 3.2046x; 3.2046x over previous
import functools

import jax
import jax.numpy as jnp
from jax.experimental import pallas as pl
from jax.experimental.pallas import tpu as pltpu

FDIM = 768
N_E = 1024
E_DIM = 256
W = 256
BETA = 0.25
N_TOK = 32 * 576
TILE = 512
N_TILES = N_TOK // TILE


def _ln(x, g, b):
    m = jnp.mean(x, axis=-1, keepdims=True)
    v = jnp.mean((x - m) ** 2, axis=-1, keepdims=True)
    return (x - m) / jnp.sqrt(v + 1e-5) * g + b


def _silu(x):
    return x * jax.nn.sigmoid(x)


def _mm(a, b):
    return jax.lax.dot_general(a, b, (((1,), (0,)), ((), ())),
                               preferred_element_type=jnp.float32)


def _prep_kernel(emb, g0, b0, w1, b1, g1, b1b, w2, b2, g2, b2b, w3, b3,
                 cb_out, dec_out):
    e = emb[...]
    n = jnp.sqrt(jnp.sum(e * e, axis=-1, keepdims=True))
    cb = e / jnp.maximum(n, 1e-12)
    cb_out[...] = cb
    x = _ln(cb, g0[...], b0[...])
    x = _silu(_mm(x, w1[...]) + b1[...])
    x = _ln(x, g1[...], b1b[...])
    x = _silu(_mm(x, w2[...]) + b2[...])
    x = _ln(x, g2[...], b2b[...])
    dec_out[...] = _mm(x, w3[...]) + b3[...]


def _main_kernel(z_ref, eg0, eb0, ew1, ebi1, eg1, ebb1, ew2, ebi2, eg2, ebb2,
                 ew3, ebi3, cb_ref, d_ref,
                 h_out, zhat_out, rec_out, commit_out, kl_out, lb_out, perp_out,
                 psum_acc, pcomp_acc, hist_acc, m_acc, c_acc, sq_acc):
    step = pl.program_id(0)

    @pl.when(step == 0)
    def _init():
        psum_acc[...] = jnp.zeros_like(psum_acc)
        pcomp_acc[...] = jnp.zeros_like(pcomp_acc)
        hist_acc[...] = jnp.zeros_like(hist_acc)
        m_acc[...] = jnp.zeros_like(m_acc)
        c_acc[...] = jnp.zeros_like(c_acc)
        sq_acc[...] = jnp.zeros_like(sq_acc)

    z = z_ref[...]
    x = _ln(z, eg0[...], eb0[...])
    x = _silu(_mm(x, ew1[...]) + ebi1[...])
    x = _ln(x, eg1[...], ebb1[...])
    x = _silu(_mm(x, ew2[...]) + ebi2[...])
    x = _ln(x, eg2[...], ebb2[...])
    h = _mm(x, ew3[...]) + ebi3[...]
    hn = jnp.sqrt(jnp.sum(h * h, axis=-1, keepdims=True))
    h = h / jnp.maximum(hn, 1e-12)
    h_out[...] = h

    cb = cb_ref[...]
    hn2 = jnp.sqrt(jnp.sum(h * h, axis=-1, keepdims=True))
    hd = h / jnp.maximum(hn2, 1e-12)
    d = jax.lax.dot_general(hd, cb, (((1,), (1,)), ((), ())),
                            preferred_element_type=jnp.float32)
    m = jnp.max(d, axis=1, keepdims=True)
    iota = jax.lax.broadcasted_iota(jnp.int32, d.shape, 1)
    idx = jnp.min(jnp.where(d == m, iota, N_E), axis=1)
    one_hot = (iota == idx[:, None]).astype(jnp.float32)
    e = jnp.exp(d - m)
    s = jnp.sum(e, axis=1, keepdims=True)
    prob = e / s
    x_new = jnp.sum(prob, axis=0)[None, :] - pcomp_acc[...]
    t = psum_acc[...] + x_new
    pcomp_acc[...] = (t - psum_acc[...]) - x_new
    psum_acc[...] = t
    hist_acc[...] += jnp.sum(one_hot, axis=0)[None, :]

    zhat = _mm(one_hot, d_ref[...])
    zhat_out[...] = zhat

    zn = jnp.sqrt(jnp.sum(z * z, axis=-1, keepdims=True))
    nzh = jnp.sqrt(jnp.sum(zhat * zhat, axis=-1, keepdims=True))
    dot = jnp.sum(zhat * z, axis=-1, keepdims=True)
    cosr = dot / (jnp.maximum(nzh, 1e-6) * jnp.maximum(zn, 1e-6))
    diff = zhat - z
    m_acc[...] += jnp.sum(m, axis=0, keepdims=True)
    c_acc[...] += jnp.sum(cosr, axis=0, keepdims=True)
    sq_acc[...] += jnp.sum(jnp.sum(diff * diff, axis=1, keepdims=True),
                           axis=0, keepdims=True)

    @pl.when(step == N_TILES - 1)
    def _finalize():
        n_tok = jnp.float32(N_TOK)
        e_mean = hist_acc[...] / n_tok
        p = psum_acc[...] / n_tok
        kl = jnp.sum(p * (jnp.log(p) - jnp.log(1.0 / N_E)),
                     axis=1, keepdims=True)
        lb = jnp.sum(e_mean * p, axis=1, keepdims=True)
        perp = jnp.exp(-jnp.sum(e_mean * jnp.log(e_mean + 1e-6),
                                axis=1, keepdims=True))
        commit = (1.0 + BETA) * (1.0 - m_acc[...] / n_tok)
        rec = (1.0 - c_acc[...] / n_tok
               + 0.001 * sq_acc[...] / (n_tok * FDIM))
        kl_out[...] = kl
        lb_out[...] = lb
        perp_out[...] = perp
        commit_out[...] = commit
        rec_out[...] = rec


def _row(v):
    return v.reshape(1, -1)


@jax.jit
def kernel(z, params):
    z_flat = z.reshape(-1, FDIM)

    dec_args = (params['emb'],
                _row(params['dec_ln0_g']), _row(params['dec_ln0_b']),
                params['dec_w1'], _row(params['dec_b1']),
                _row(params['dec_ln1_g']), _row(params['dec_ln1_b']),
                params['dec_w2'], _row(params['dec_b2']),
                _row(params['dec_ln2_g']), _row(params['dec_ln2_b']),
                params['dec_w3'], _row(params['dec_b3']))
    cb, dec_tab = pl.pallas_call(
        _prep_kernel,
        out_shape=(jax.ShapeDtypeStruct((N_E, E_DIM), jnp.float32),
                   jax.ShapeDtypeStruct((N_E, FDIM), jnp.float32)),
    )(*dec_args)

    enc_args = (_row(params['enc_ln0_g']), _row(params['enc_ln0_b']),
                params['enc_w1'], _row(params['enc_b1']),
                _row(params['enc_ln1_g']), _row(params['enc_ln1_b']),
                params['enc_w2'], _row(params['enc_b2']),
                _row(params['enc_ln2_g']), _row(params['enc_ln2_b']),
                params['enc_w3'], _row(params['enc_b3']))

    full = lambda a: pl.BlockSpec(a.shape, lambda i: (0, 0))
    in_specs = [pl.BlockSpec((TILE, FDIM), lambda i: (i, 0))]
    for a in enc_args:
        in_specs.append(full(a))
    in_specs.append(pl.BlockSpec((N_E, E_DIM), lambda i: (0, 0)))
    in_specs.append(pl.BlockSpec((N_E, FDIM), lambda i: (0, 0)))

    scal = jax.ShapeDtypeStruct((1, 1), jnp.float32)
    scal_spec = pl.BlockSpec((1, 1), lambda i: (0, 0))
    out_shape = (jax.ShapeDtypeStruct((N_TOK, E_DIM), jnp.float32),
                 jax.ShapeDtypeStruct((N_TOK, FDIM), jnp.float32),
                 scal, scal, scal, scal, scal)
    out_specs = (pl.BlockSpec((TILE, E_DIM), lambda i: (i, 0)),
                 pl.BlockSpec((TILE, FDIM), lambda i: (i, 0)),
                 scal_spec, scal_spec, scal_spec, scal_spec, scal_spec)

    h, zhat, rec, commit, kl, lb, perp = pl.pallas_call(
        _main_kernel,
        grid=(N_TILES,),
        in_specs=in_specs,
        out_specs=out_specs,
        out_shape=out_shape,
        scratch_shapes=[pltpu.VMEM((1, N_E), jnp.float32),
                        pltpu.VMEM((1, N_E), jnp.float32),
                        pltpu.VMEM((1, N_E), jnp.float32),
                        pltpu.VMEM((1, 1), jnp.float32),
                        pltpu.VMEM((1, 1), jnp.float32),
                        pltpu.VMEM((1, 1), jnp.float32)],
        compiler_params=pltpu.CompilerParams(
            dimension_semantics=("arbitrary",)),
    )(z_flat, *enc_args, cb, dec_tab)

    return (zhat, rec[0, 0], commit[0, 0], kl[0, 0], lb[0, 0], cb, h,
            perp[0, 0])
